# baseline (speedup 1.0000x reference)
import jax
import jax.numpy as jnp
from jax.experimental import pallas as pl
from jax.experimental.pallas import tpu as pltpu

_CHUNK = 1000


def _sigmoid_block(a_ref, b_ref, o_ref):
    o_ref[0:_CHUNK, :] = jax.nn.sigmoid(a_ref[...])
    o_ref[_CHUNK : 2 * _CHUNK, :] = jax.nn.sigmoid(b_ref[...])


def kernel(x, edge_index):
    del edge_index
    n_rows, d = x.shape
    grid = (n_rows // (2 * _CHUNK),)
    return pl.pallas_call(
        _sigmoid_block,
        grid=grid,
        in_specs=[
            pl.BlockSpec((_CHUNK, d), lambda i: (2 * i, 0)),
            pl.BlockSpec((_CHUNK, d), lambda i: (2 * i + 1, 0)),
        ],
        out_specs=pl.BlockSpec((2 * _CHUNK, d), lambda i: (i, 0)),
        out_shape=jax.ShapeDtypeStruct(x.shape, x.dtype),
    )(x, x)


# speedup vs baseline: 1.4760x; 1.4760x over previous
import jax
import jax.numpy as jnp
from jax.experimental import pallas as pl
from jax.experimental.pallas import tpu as pltpu

_CHUNK = 1000


def _sigmoid_block(a_ref, b_ref, c_ref, d_ref, e_ref, o_ref):
    for k, r in enumerate((a_ref, b_ref, c_ref, d_ref, e_ref)):
        o_ref[k * _CHUNK : (k + 1) * _CHUNK, :] = jax.nn.sigmoid(r[...])


def kernel(x, edge_index):
    del edge_index
    n_rows, d = x.shape
    grid = (n_rows // (5 * _CHUNK),)
    return pl.pallas_call(
        _sigmoid_block,
        grid=grid,
        in_specs=[
            pl.BlockSpec((_CHUNK, d), lambda i, k=k: (5 * i + k, 0))
            for k in range(5)
        ],
        out_specs=pl.BlockSpec((5 * _CHUNK, d), lambda i: (i, 0)),
        out_shape=jax.ShapeDtypeStruct(x.shape, x.dtype),
    )(x, x, x, x, x)


# speedup vs baseline: 1.7368x; 1.1767x over previous
import jax
import jax.numpy as jnp
from jax.experimental import pallas as pl
from jax.experimental.pallas import tpu as pltpu

_NCHUNK = 10
_CHUNK = 1000


def _sigmoid_manual(x_hbm, o_hbm, x_vmem, o_vmem, in_sems, out_sems):
    in_copies = []
    for c in range(_NCHUNK):
        sl = pl.ds(c * _CHUNK, _CHUNK)
        cp = pltpu.make_async_copy(
            x_hbm.at[sl, :], x_vmem.at[sl, :], in_sems.at[c]
        )
        cp.start()
        in_copies.append(cp)
    out_copies = []
    for c in range(_NCHUNK):
        sl = pl.ds(c * _CHUNK, _CHUNK)
        in_copies[c].wait()
        o_vmem[sl, :] = jax.nn.sigmoid(x_vmem[sl, :])
        cp = pltpu.make_async_copy(
            o_vmem.at[sl, :], o_hbm.at[sl, :], out_sems.at[c]
        )
        cp.start()
        out_copies.append(cp)
    for cp in out_copies:
        cp.wait()


def kernel(x, edge_index):
    del edge_index
    n_rows, d = x.shape
    return pl.pallas_call(
        _sigmoid_manual,
        in_specs=[pl.BlockSpec(memory_space=pltpu.MemorySpace.HBM)],
        out_specs=pl.BlockSpec(memory_space=pltpu.MemorySpace.HBM),
        out_shape=jax.ShapeDtypeStruct(x.shape, x.dtype),
        scratch_shapes=[
            pltpu.VMEM((n_rows, d), jnp.float32),
            pltpu.VMEM((n_rows, d), jnp.float32),
            pltpu.SemaphoreType.DMA((_NCHUNK,)),
            pltpu.SemaphoreType.DMA((_NCHUNK,)),
        ],
    )(x)
